# baseline (speedup 1.0000x reference)
import functools

import jax
import jax.numpy as jnp
from jax import lax
from jax.experimental import pallas as pl
from jax.experimental.pallas import tpu as pltpu

_EPS = 1e-5
_MIB = 1024 * 1024
_INF = jnp.inf


def _gate_body(x_ref, g_ref, wx_ref, wg_ref, pc_ref, out_ref, *, inv_n, s,
               F_int):
    x = x_ref[0]
    g = g_ref[0]
    pc = pc_ref[...]
    bx, bg = pc[:, 0:1], pc[:, 1:2]
    wpx, wpg = pc[:, 2:3], pc[:, 3:4]
    dbase = pc[:, 4:5]
    wpsi = pc[:, 5:6]
    bpsi = pc[0, 6]

    xa = jnp.dot(wx_ref[...], x, preferred_element_type=jnp.float32)
    ga = jnp.dot(wg_ref[...], g, preferred_element_type=jnp.float32)
    x1 = xa[:F_int]
    g1 = ga[:F_int]

    sum_x = jnp.sum(xa[F_int]) + s * jnp.sum(bx)
    crs_x = jnp.sum(xa[F_int + 1])
    sum_g = jnp.sum(ga[F_int]) + s * jnp.sum(bg)
    crs_g = jnp.sum(ga[F_int + 1])

    ssq_x = jnp.sum(x1 * x1) + 2.0 * crs_x + s * jnp.sum(bx * bx)
    ssq_g = jnp.sum(g1 * g1) + 2.0 * crs_g + s * jnp.sum(bg * bg)

    mu_x = sum_x * inv_n
    var_x = jnp.maximum(ssq_x * inv_n - mu_x * mu_x, 0.0)
    mu_g = sum_g * inv_n
    var_g = jnp.maximum(ssq_g * inv_n - mu_g * mu_g, 0.0)

    ax = wpx * lax.rsqrt(var_x + _EPS)
    ag = wpg * lax.rsqrt(var_g + _EPS)
    cc = ax * (bx - mu_x) + ag * (bg - mu_g) + dbase
    lo = jnp.where(wpsi >= 0.0, 0.0, -_INF)
    hi = jnp.where(wpsi >= 0.0, _INF, 0.0)

    u = x1 * ax + g1 * ag + cc
    contrib = jnp.clip(u, lo, hi)
    psi = jnp.sum(contrib, axis=0, keepdims=True)
    gate = jax.nn.sigmoid(psi + bpsi)
    out_ref[0] = x * gate


def _augment(w, b, pad_to):
    f_out, f_in = w.shape
    w32 = w.astype(jnp.float32)
    extra = jnp.concatenate(
        [jnp.sum(w32, axis=0, keepdims=True),
         b.reshape(1, f_out).astype(jnp.float32) @ w32], axis=0)
    aug = jnp.concatenate([w32, extra], axis=0)
    return jnp.pad(aug, ((0, pad_to - f_out - 2), (0, 0)))


def kernel(x, g, wx, bx, gx_w, gx_b, wg, bg, gg_w, gg_b, wpsi, bpsi):
    N, F_l, H, W = x.shape
    F_g = g.shape[1]
    F_int = wx.shape[0]
    S = H * W
    f32 = jnp.float32
    pad = F_int + 8

    xr = x.reshape(N, F_l, S)
    gr = g.reshape(N, F_g, S)
    wxa = _augment(wx, bx, pad)
    wga = _augment(wg, bg, pad)

    col = lambda a: a.reshape(F_int, 1).astype(f32)
    wpsi_c = col(wpsi)
    wpx = wpsi_c * col(gx_w)
    wpg = wpsi_c * col(gg_w)
    dbase = wpsi_c * (col(gx_b) + col(gg_b))
    bpsi_c = jnp.full((F_int, 1), bpsi.reshape(()).astype(f32))
    zero = jnp.zeros((F_int, 1), f32)
    pc = jnp.concatenate(
        [col(bx), col(bg), wpx, wpg, dbase, wpsi_c, bpsi_c, zero], axis=1)

    whole = lambda shape: pl.BlockSpec(shape, lambda b: (0, 0))
    out = pl.pallas_call(
        functools.partial(_gate_body, inv_n=1.0 / float(F_int * S),
                          s=float(S), F_int=F_int),
        out_shape=jax.ShapeDtypeStruct((N, F_l, S), x.dtype),
        grid=(N,),
        in_specs=[
            pl.BlockSpec((1, F_l, S), lambda b: (b, 0, 0)),
            pl.BlockSpec((1, F_g, S), lambda b: (b, 0, 0)),
            whole((pad, F_l)),
            whole((pad, F_g)),
            whole((F_int, 8)),
        ],
        out_specs=pl.BlockSpec((1, F_l, S), lambda b: (b, 0, 0)),
        compiler_params=pltpu.CompilerParams(
            dimension_semantics=("parallel",),
            vmem_limit_bytes=56 * _MIB),
    )(xr, gr, wxa, wga, pc)
    return out.reshape(N, F_l, H, W)


# speedup vs baseline: 1.0592x; 1.0592x over previous
import functools

import jax
import jax.numpy as jnp
from jax import lax
from jax.experimental import pallas as pl
from jax.experimental.pallas import tpu as pltpu

_EPS = 1e-5
_MIB = 1024 * 1024
_INF = jnp.inf


def _moments(a):
    s = jnp.sum(a, axis=0, keepdims=True)
    q = jnp.sum(a * a, axis=0, keepdims=True)
    return (jnp.sum(s, axis=1, keepdims=True),
            jnp.sum(q, axis=1, keepdims=True))


def _gate_body(x_ref, g_ref, wx_ref, wg_ref, pc_ref, out_ref, *, inv_n):
    x = x_ref[0]
    g = g_ref[0]
    pc = pc_ref[...]
    bx, bg = pc[:, 0:1], pc[:, 1:2]
    wpx, wpg = pc[:, 2:3], pc[:, 3:4]
    dbase = pc[:, 4:5]
    wpsi = pc[:, 5:6]
    bpsi = pc[0:1, 6:7]

    x1 = jnp.dot(wx_ref[...], x, preferred_element_type=jnp.float32) + bx
    g1 = jnp.dot(wg_ref[...], g, preferred_element_type=jnp.float32) + bg

    sum_x, ssq_x = _moments(x1)
    sum_g, ssq_g = _moments(g1)

    mu_x = sum_x * inv_n
    var_x = jnp.maximum(ssq_x * inv_n - mu_x * mu_x, 0.0)
    mu_g = sum_g * inv_n
    var_g = jnp.maximum(ssq_g * inv_n - mu_g * mu_g, 0.0)

    ax = wpx * lax.rsqrt(var_x + _EPS)
    ag = wpg * lax.rsqrt(var_g + _EPS)
    cc = dbase - ax * mu_x - ag * mu_g
    lo = jnp.where(wpsi >= 0.0, 0.0, -_INF)
    hi = jnp.where(wpsi >= 0.0, _INF, 0.0)

    u = x1 * ax + g1 * ag + cc
    contrib = jnp.clip(u, lo, hi)
    psi = jnp.sum(contrib, axis=0, keepdims=True)
    gate = jax.nn.sigmoid(psi + bpsi)
    out_ref[0] = x * gate


def kernel(x, g, wx, bx, gx_w, gx_b, wg, bg, gg_w, gg_b, wpsi, bpsi):
    N, F_l, H, W = x.shape
    F_g = g.shape[1]
    F_int = wx.shape[0]
    S = H * W
    f32 = jnp.float32

    xr = x.reshape(N, F_l, S)
    gr = g.reshape(N, F_g, S)

    col = lambda a: a.reshape(F_int, 1).astype(f32)
    wpsi_c = col(wpsi)
    wpx = wpsi_c * col(gx_w)
    wpg = wpsi_c * col(gg_w)
    dbase = wpsi_c * (col(gx_b) + col(gg_b) - col(bx) * 0.0)
    dbase = dbase + wpx * col(bx) * 0.0
    bpsi_c = jnp.full((F_int, 1), bpsi.reshape(()).astype(f32))
    zero = jnp.zeros((F_int, 1), f32)
    pc = jnp.concatenate(
        [col(bx), col(bg), wpx, wpg, dbase, wpsi_c, bpsi_c, zero], axis=1)

    whole = lambda shape: pl.BlockSpec(shape, lambda b: (0, 0))
    out = pl.pallas_call(
        functools.partial(_gate_body, inv_n=1.0 / float(F_int * S)),
        out_shape=jax.ShapeDtypeStruct((N, F_l, S), x.dtype),
        grid=(N,),
        in_specs=[
            pl.BlockSpec((1, F_l, S), lambda b: (b, 0, 0)),
            pl.BlockSpec((1, F_g, S), lambda b: (b, 0, 0)),
            whole((F_int, F_l)),
            whole((F_int, F_g)),
            whole((F_int, 8)),
        ],
        out_specs=pl.BlockSpec((1, F_l, S), lambda b: (b, 0, 0)),
        compiler_params=pltpu.CompilerParams(
            dimension_semantics=("parallel",),
            vmem_limit_bytes=56 * _MIB),
    )(xr, gr, wx.astype(f32), wg.astype(f32), pc)
    return out.reshape(N, F_l, H, W)


# speedup vs baseline: 1.0913x; 1.0303x over previous
import functools

import jax
import jax.numpy as jnp
from jax import lax
from jax.experimental import pallas as pl
from jax.experimental.pallas import tpu as pltpu

_EPS = 1e-5
_MIB = 1024 * 1024
_INF = jnp.inf
_DEPTH = 3


def _moments(a):
    s = jnp.sum(a, axis=0, keepdims=True)
    q = jnp.sum(a * a, axis=0, keepdims=True)
    return (jnp.sum(s, axis=1, keepdims=True),
            jnp.sum(q, axis=1, keepdims=True))


def _gate_body(x_hbm, g_hbm, wx_ref, wg_ref, pc_ref, out_hbm,
               xb, gb, ob, sem_x, sem_g, sem_o, *, inv_n, N):
    b = pl.program_id(0)
    slot = lax.rem(b, _DEPTH)
    oslot = lax.rem(b, 2)

    @pl.when(b == 0)
    def _prologue():
        for d in range(min(_DEPTH, N)):
            pltpu.make_async_copy(x_hbm.at[d], xb.at[d], sem_x.at[d]).start()
            pltpu.make_async_copy(g_hbm.at[d], gb.at[d], sem_g.at[d]).start()

    pltpu.make_async_copy(x_hbm.at[b], xb.at[slot], sem_x.at[slot]).wait()
    pltpu.make_async_copy(g_hbm.at[b], gb.at[slot], sem_g.at[slot]).wait()

    x = xb[slot]
    g = gb[slot]
    pc = pc_ref[...]
    bx, bg = pc[:, 0:1], pc[:, 1:2]
    wpx, wpg = pc[:, 2:3], pc[:, 3:4]
    dbase = pc[:, 4:5]
    wpsi = pc[:, 5:6]
    bpsi = pc[0:1, 6:7]

    x1 = jnp.dot(wx_ref[...], x, preferred_element_type=jnp.float32) + bx
    g1 = jnp.dot(wg_ref[...], g, preferred_element_type=jnp.float32) + bg

    sum_x, ssq_x = _moments(x1)
    sum_g, ssq_g = _moments(g1)

    mu_x = sum_x * inv_n
    var_x = jnp.maximum(ssq_x * inv_n - mu_x * mu_x, 0.0)
    mu_g = sum_g * inv_n
    var_g = jnp.maximum(ssq_g * inv_n - mu_g * mu_g, 0.0)

    ax = wpx * lax.rsqrt(var_x + _EPS)
    ag = wpg * lax.rsqrt(var_g + _EPS)
    cc = dbase - ax * mu_x - ag * mu_g
    lo = jnp.where(wpsi >= 0.0, 0.0, -_INF)
    hi = jnp.where(wpsi >= 0.0, _INF, 0.0)

    @pl.when(b >= 2)
    def _wait_oslot():
        pltpu.make_async_copy(ob.at[oslot], out_hbm.at[b - 2],
                              sem_o.at[oslot]).wait()

    u = x1 * ax + g1 * ag + cc
    contrib = jnp.clip(u, lo, hi)
    psi = jnp.sum(contrib, axis=0, keepdims=True)
    gate = jax.nn.sigmoid(psi + bpsi)
    ob[oslot] = x * gate

    pltpu.make_async_copy(ob.at[oslot], out_hbm.at[b], sem_o.at[oslot]).start()

    @pl.when(b + _DEPTH < N)
    def _refill():
        nb = b + _DEPTH
        pltpu.make_async_copy(x_hbm.at[nb], xb.at[slot], sem_x.at[slot]).start()
        pltpu.make_async_copy(g_hbm.at[nb], gb.at[slot], sem_g.at[slot]).start()

    @pl.when(b == N - 1)
    def _drain():
        if N >= 2:
            pltpu.make_async_copy(ob.at[(N - 2) % 2], out_hbm.at[N - 2],
                                  sem_o.at[(N - 2) % 2]).wait()
        pltpu.make_async_copy(ob.at[(N - 1) % 2], out_hbm.at[N - 1],
                              sem_o.at[(N - 1) % 2]).wait()


def kernel(x, g, wx, bx, gx_w, gx_b, wg, bg, gg_w, gg_b, wpsi, bpsi):
    N, F_l, H, W = x.shape
    F_g = g.shape[1]
    F_int = wx.shape[0]
    S = H * W
    f32 = jnp.float32

    xr = x.reshape(N, F_l, S)
    gr = g.reshape(N, F_g, S)

    col = lambda a: a.reshape(F_int, 1).astype(f32)
    wpsi_c = col(wpsi)
    wpx = wpsi_c * col(gx_w)
    wpg = wpsi_c * col(gg_w)
    dbase = wpsi_c * (col(gx_b) + col(gg_b))
    bpsi_c = jnp.full((F_int, 1), bpsi.reshape(()).astype(f32))
    zero = jnp.zeros((F_int, 1), f32)
    pc = jnp.concatenate(
        [col(bx), col(bg), wpx, wpg, dbase, wpsi_c, bpsi_c, zero], axis=1)

    whole = lambda shape: pl.BlockSpec(shape, lambda b: (0, 0))
    out = pl.pallas_call(
        functools.partial(_gate_body, inv_n=1.0 / float(F_int * S), N=N),
        out_shape=jax.ShapeDtypeStruct((N, F_l, S), x.dtype),
        grid=(N,),
        in_specs=[
            pl.BlockSpec(memory_space=pl.ANY),
            pl.BlockSpec(memory_space=pl.ANY),
            whole((F_int, F_l)),
            whole((F_int, F_g)),
            whole((F_int, 8)),
        ],
        out_specs=pl.BlockSpec(memory_space=pl.ANY),
        scratch_shapes=[
            pltpu.VMEM((_DEPTH, F_l, S), f32),
            pltpu.VMEM((_DEPTH, F_g, S), f32),
            pltpu.VMEM((2, F_l, S), f32),
            pltpu.SemaphoreType.DMA((_DEPTH,)),
            pltpu.SemaphoreType.DMA((_DEPTH,)),
            pltpu.SemaphoreType.DMA((2,)),
        ],
        compiler_params=pltpu.CompilerParams(
            dimension_semantics=("arbitrary",),
            vmem_limit_bytes=56 * _MIB),
    )(xr, gr, wx.astype(f32), wg.astype(f32), pc)
    return out.reshape(N, F_l, H, W)
